# Initial kernel scaffold; baseline (speedup 1.0000x reference)
import jax
import jax.numpy as jnp
from jax.experimental import pallas as pl


def kernel(ent_pkl, other_emb, W_proj, batch_input_seqs, target_ent_index):
    raise NotImplementedError("write your pallas kernel here")


# speedup vs baseline: 1.8390x; 1.8390x over previous
import functools

import jax
import jax.numpy as jnp
from jax import lax
from jax.experimental import pallas as pl
from jax.experimental.pallas import tpu as pltpu
from jax.experimental.pallas import tpu_sc as plsc

_NUM_ENT = 100000
_NUM_REL = 200
_D = 64
_F = 128
_B = 1024
_NIDX = 3 * _B
_NQ = 2 * _B


def _sc_gather(ent_pkl, other_emb, feat_idx, spec_idx):
    info = plsc.get_sparse_core_info()
    nc, ns = info.num_cores, info.num_subcores
    nw = nc * ns
    bpw_f = _NIDX // nw
    bpw_s = _NQ // nw
    mesh = plsc.VectorSubcoreMesh(core_axis_name="c", subcore_axis_name="s")

    @functools.partial(
        pl.kernel,
        mesh=mesh,
        out_type=(
            jax.ShapeDtypeStruct((_NIDX, _F), jnp.float32),
            jax.ShapeDtypeStruct((_NQ, _F), jnp.float32),
        ),
        scratch_types=[
            pltpu.VMEM((bpw_f,), jnp.int32),
            pltpu.VMEM((bpw_f, _F), jnp.float32),
            pltpu.VMEM((bpw_s,), jnp.int32),
            pltpu.VMEM((bpw_s, _F), jnp.float32),
            pltpu.SemaphoreType.DMA,
            pltpu.SemaphoreType.DMA,
        ],
    )
    def gather_k(ent_hbm, other_hbm, fidx_hbm, sidx_hbm, feats_out, specs_out,
                 fidx_v, frows_v, sidx_v, srows_v, sem_f, sem_s):
        wid = lax.axis_index("s") * nc + lax.axis_index("c")
        base_f = wid * bpw_f
        base_s = wid * bpw_s
        pltpu.sync_copy(fidx_hbm.at[pl.ds(base_f, bpw_f)], fidx_v)
        pltpu.sync_copy(sidx_hbm.at[pl.ds(base_s, bpw_s)], sidx_v)
        cp_f = pltpu.async_copy(ent_hbm.at[fidx_v], frows_v, sem_f)
        cp_s = pltpu.async_copy(other_hbm.at[sidx_v], srows_v, sem_s)
        cp_f.wait()
        cp_s.wait()
        pltpu.sync_copy(frows_v, feats_out.at[pl.ds(base_f, bpw_f)])
        pltpu.sync_copy(srows_v, specs_out.at[pl.ds(base_s, bpw_s)])

    return gather_k(ent_pkl, other_emb, feat_idx, spec_idx)


def _prep_body(feats_ref, w_ref, specs_ref, mask_q_ref, mask_tT_ref,
               other0_ref, qs_ref, teT_ref):
    w = w_ref[...]
    proj_q = jnp.dot(feats_ref[:_NQ, :], w,
                     preferred_element_type=jnp.float32)
    mq = mask_q_ref[...]
    rows_q = mq * proj_q + (1.0 - mq) * specs_ref[:, :_D]
    nrm = jnp.sqrt(jnp.sum(rows_q * rows_q, axis=-1, keepdims=True))
    qn = rows_q / jnp.maximum(nrm, 1e-12)
    qs_ref[...] = qn[:_B] + qn[_B:]
    projT_t = lax.dot_general(
        w, feats_ref[_NQ:, :], (((0,), (1,)), ((), ())),
        preferred_element_type=jnp.float32)
    mt = mask_tT_ref[...]
    teT = mt * projT_t + (1.0 - mt) * other0_ref[...]
    teT_ref[...] = teT.astype(jnp.bfloat16)


_BI = 128
_BJ = 128


_NBI = _B // _BI
_NBJ = _B // _BJ
_DPG = _D // _NBJ


def _score_body(qs_ref, teT_ref, out_ref, qb_ref):
    i = pl.program_id(0)
    j = pl.program_id(1)

    @pl.when(j == 0)
    def _():
        q = qs_ref[pl.ds(i * _BI, _BI), :].astype(jnp.bfloat16)
        for d in range(_D):
            qb_ref[d] = jnp.broadcast_to(q[:, d:d + 1], (_BI, _BJ))

    accs = [jnp.zeros((_BI, _BJ), jnp.bfloat16) for _ in range(4)]
    for d in range(_D):
        accs[d % 4] = accs[d % 4] + jnp.abs(qb_ref[d] - teT_ref[d:d + 1, :])
    s1 = [accs[0] + accs[1], accs[2] + accs[3]]
    out_ref[...] = -(s1[0] + s1[1]).astype(jnp.float32)


def kernel(ent_pkl, other_emb, W_proj, batch_input_seqs, target_ent_index):
    seq = batch_input_seqs.astype(jnp.int32)
    t_idx = target_ent_index.astype(jnp.int32)
    all_idx = jnp.concatenate([seq[:, 0], seq[:, 1], t_idx])
    is_ent = (all_idx >= 1) & (all_idx <= _NUM_ENT)
    feat_idx = jnp.where(is_ent, all_idx - 1, 0).astype(jnp.int32)
    q_idx = all_idx[:_NQ]
    spec_idx = jnp.where(q_idx == 0, 0, q_idx - _NUM_ENT)
    spec_idx = jnp.clip(spec_idx, 0, _NUM_REL + 2).astype(jnp.int32)

    other_pad = jnp.pad(other_emb, ((0, 0), (0, _F - _D)))
    feats, specs = _sc_gather(ent_pkl, other_pad, feat_idx, spec_idx)

    mask = is_ent.astype(jnp.float32)
    mask_q = mask[:_NQ, None]
    mask_tT = mask[None, _NQ:]
    other0 = other_emb[0][:, None]

    qs, teT = pl.pallas_call(
        _prep_body,
        out_shape=(
            jax.ShapeDtypeStruct((_B, _D), jnp.float32),
            jax.ShapeDtypeStruct((_D, _B), jnp.bfloat16),
        ),
    )(feats, W_proj, specs, mask_q, mask_tT, other0)

    score = pl.pallas_call(
        _score_body,
        grid=(_B // _BI, _B // _BJ),
        in_specs=[
            pl.BlockSpec((_B, _D), lambda i, j: (0, 0)),
            pl.BlockSpec((_D, _BJ), lambda i, j: (0, j)),
        ],
        out_specs=pl.BlockSpec((_BI, _BJ), lambda i, j: (i, j)),
        out_shape=jax.ShapeDtypeStruct((_B, _B), jnp.float32),
        scratch_shapes=[pltpu.VMEM((2 * _D, _BI, _BJ), jnp.bfloat16)],
    )(qs, teT)
    return score


# speedup vs baseline: 1.8441x; 1.0028x over previous
import functools

import jax
import jax.numpy as jnp
from jax import lax
from jax.experimental import pallas as pl
from jax.experimental.pallas import tpu as pltpu
from jax.experimental.pallas import tpu_sc as plsc

_NUM_ENT = 100000
_NUM_REL = 200
_D = 64
_F = 128
_B = 1024
_NIDX = 3 * _B
_NQ = 2 * _B


def _sc_gather(ent_pkl, other_emb, feat_idx, spec_idx):
    info = plsc.get_sparse_core_info()
    nc, ns = info.num_cores, info.num_subcores
    nw = nc * ns
    bpw_f = _NIDX // nw
    bpw_s = _NQ // nw
    mesh = plsc.VectorSubcoreMesh(core_axis_name="c", subcore_axis_name="s")

    @functools.partial(
        pl.kernel,
        mesh=mesh,
        compiler_params=pltpu.CompilerParams(use_tc_tiling_on_sc=True),
        out_type=(
            jax.ShapeDtypeStruct((_NIDX, _F), jnp.float32),
            jax.ShapeDtypeStruct((_NQ, _F), jnp.float32),
        ),
        scratch_types=[
            pltpu.VMEM((bpw_f,), jnp.int32),
            pltpu.VMEM((bpw_f, _F), jnp.float32),
            pltpu.VMEM((bpw_s,), jnp.int32),
            pltpu.VMEM((bpw_s, _F), jnp.float32),
            pltpu.SemaphoreType.DMA,
            pltpu.SemaphoreType.DMA,
        ],
    )
    def gather_k(ent_hbm, other_hbm, fidx_hbm, sidx_hbm, feats_out, specs_out,
                 fidx_v, frows_v, sidx_v, srows_v, sem_f, sem_s):
        wid = lax.axis_index("s") * nc + lax.axis_index("c")
        base_f = wid * bpw_f
        base_s = wid * bpw_s
        pltpu.sync_copy(fidx_hbm.at[pl.ds(base_f, bpw_f)], fidx_v)
        pltpu.sync_copy(sidx_hbm.at[pl.ds(base_s, bpw_s)], sidx_v)
        cp_f = pltpu.async_copy(ent_hbm.at[fidx_v], frows_v, sem_f)
        cp_s = pltpu.async_copy(other_hbm.at[sidx_v], srows_v, sem_s)
        cp_f.wait()
        cp_s.wait()
        pltpu.sync_copy(frows_v, feats_out.at[pl.ds(base_f, bpw_f)])
        pltpu.sync_copy(srows_v, specs_out.at[pl.ds(base_s, bpw_s)])

    return gather_k(ent_pkl, other_emb, feat_idx, spec_idx)


def _prep_body(feats_ref, w_ref, specs_ref, mask_q_ref, mask_tT_ref,
               other0_ref, qs_ref, teT_ref):
    w = w_ref[...]
    proj_q = jnp.dot(feats_ref[:_NQ, :], w,
                     preferred_element_type=jnp.float32)
    mq = mask_q_ref[...]
    rows_q = mq * proj_q + (1.0 - mq) * specs_ref[:, :_D]
    nrm = jnp.sqrt(jnp.sum(rows_q * rows_q, axis=-1, keepdims=True))
    qn = rows_q / jnp.maximum(nrm, 1e-12)
    qs_ref[...] = qn[:_B] + qn[_B:]
    projT_t = lax.dot_general(
        w, feats_ref[_NQ:, :], (((0,), (1,)), ((), ())),
        preferred_element_type=jnp.float32)
    mt = mask_tT_ref[...]
    teT = mt * projT_t + (1.0 - mt) * other0_ref[...]
    teT_ref[...] = teT.astype(jnp.bfloat16)


_BI = 128
_BJ = 128


_NBI = _B // _BI
_NBJ = _B // _BJ
_DPG = _D // _NBJ


def _score_body(qs_ref, teT_ref, out_ref, qb_ref):
    i = pl.program_id(0)
    j = pl.program_id(1)

    @pl.when(j == 0)
    def _():
        q = qs_ref[pl.ds(i * _BI, _BI), :].astype(jnp.bfloat16)
        for d in range(_D):
            qb_ref[d] = jnp.broadcast_to(q[:, d:d + 1], (_BI, _BJ))

    accs = [jnp.zeros((_BI, _BJ), jnp.bfloat16) for _ in range(4)]
    for d in range(_D):
        accs[d % 4] = accs[d % 4] + jnp.abs(qb_ref[d] - teT_ref[d:d + 1, :])
    s1 = [accs[0] + accs[1], accs[2] + accs[3]]
    out_ref[...] = -(s1[0] + s1[1]).astype(jnp.float32)


def kernel(ent_pkl, other_emb, W_proj, batch_input_seqs, target_ent_index):
    seq = batch_input_seqs.astype(jnp.int32)
    t_idx = target_ent_index.astype(jnp.int32)
    all_idx = jnp.concatenate([seq[:, 0], seq[:, 1], t_idx])
    is_ent = (all_idx >= 1) & (all_idx <= _NUM_ENT)
    feat_idx = jnp.where(is_ent, all_idx - 1, 0).astype(jnp.int32)
    q_idx = all_idx[:_NQ]
    spec_idx = jnp.where(q_idx == 0, 0, q_idx - _NUM_ENT)
    spec_idx = jnp.clip(spec_idx, 0, _NUM_REL + 2).astype(jnp.int32)

    other_pad = jnp.pad(other_emb, ((0, 0), (0, _F - _D)))
    feats, specs = _sc_gather(ent_pkl, other_pad, feat_idx, spec_idx)

    mask = is_ent.astype(jnp.float32)
    mask_q = mask[:_NQ, None]
    mask_tT = mask[None, _NQ:]
    other0 = other_emb[0][:, None]

    qs, teT = pl.pallas_call(
        _prep_body,
        out_shape=(
            jax.ShapeDtypeStruct((_B, _D), jnp.float32),
            jax.ShapeDtypeStruct((_D, _B), jnp.bfloat16),
        ),
    )(feats, W_proj, specs, mask_q, mask_tT, other0)

    score = pl.pallas_call(
        _score_body,
        grid=(_B // _BI, _B // _BJ),
        in_specs=[
            pl.BlockSpec((_B, _D), lambda i, j: (0, 0)),
            pl.BlockSpec((_D, _BJ), lambda i, j: (0, j)),
        ],
        out_specs=pl.BlockSpec((_BI, _BJ), lambda i, j: (i, j)),
        out_shape=jax.ShapeDtypeStruct((_B, _B), jnp.float32),
        scratch_shapes=[pltpu.VMEM((2 * _D, _BI, _BJ), jnp.bfloat16)],
    )(qs, teT)
    return score


# speedup vs baseline: 3.6853x; 1.9984x over previous
import functools

import jax
import jax.numpy as jnp
from jax import lax
from jax.experimental import pallas as pl
from jax.experimental.pallas import tpu as pltpu
from jax.experimental.pallas import tpu_sc as plsc

_NUM_ENT = 100000
_NUM_REL = 200
_D = 64
_F = 128
_B = 1024
_NIDX = 3 * _B
_NQ = 2 * _B
_NSPAD = 208


def _sc_gather(ent_pkl, feat_idx):
    info = plsc.get_sparse_core_info()
    nc, ns = info.num_cores, info.num_subcores
    nw = nc * ns
    bpw_f = _NIDX // nw
    mesh = plsc.VectorSubcoreMesh(core_axis_name="c", subcore_axis_name="s")

    @functools.partial(
        pl.kernel,
        mesh=mesh,
        compiler_params=pltpu.CompilerParams(use_tc_tiling_on_sc=True),
        out_type=jax.ShapeDtypeStruct((_NIDX, _F), jnp.float32),
        scratch_types=[
            pltpu.VMEM((bpw_f,), jnp.int32),
            pltpu.VMEM((bpw_f, _F), jnp.float32),
            pltpu.SemaphoreType.DMA,
        ],
    )
    def gather_k(ent_hbm, fidx_hbm, feats_out, fidx_v, frows_v, sem_f):
        wid = lax.axis_index("s") * nc + lax.axis_index("c")
        base_f = wid * bpw_f
        pltpu.sync_copy(fidx_hbm.at[pl.ds(base_f, bpw_f)], fidx_v)
        pltpu.async_copy(ent_hbm.at[fidx_v], frows_v, sem_f).wait()
        pltpu.sync_copy(frows_v, feats_out.at[pl.ds(base_f, bpw_f)])

    return gather_k(ent_pkl, feat_idx)


def _prep_body(feats_ref, w_ref, other_sm_ref, spec_f_ref, mask_q_ref,
               mask_tT_ref, other0_ref, qs_ref, teT_ref):
    w = w_ref[...]
    sid = spec_f_ref[...]
    lanes = lax.broadcasted_iota(jnp.int32, (1, _NSPAD), 1)
    onehot = (sid == lanes).astype(jnp.float32)
    specs = jnp.dot(onehot, other_sm_ref[...],
                    preferred_element_type=jnp.float32)
    proj_q = jnp.dot(feats_ref[:_NQ, :], w,
                     preferred_element_type=jnp.float32)
    mq = mask_q_ref[...]
    rows_q = mq * proj_q + (1.0 - mq) * specs
    nrm = jnp.sqrt(jnp.sum(rows_q * rows_q, axis=-1, keepdims=True))
    qn = rows_q / jnp.maximum(nrm, 1e-12)
    qs_ref[...] = qn[:_B] + qn[_B:]
    projT_t = lax.dot_general(
        w, feats_ref[_NQ:, :], (((0,), (1,)), ((), ())),
        preferred_element_type=jnp.float32)
    mt = mask_tT_ref[...]
    teT = mt * projT_t + (1.0 - mt) * other0_ref[...]
    teT_ref[...] = teT.astype(jnp.bfloat16)


_BI = 128
_BJ = 128


_NBI = _B // _BI
_NBJ = _B // _BJ
_DPG = _D // _NBJ


def _score_body(qs_ref, teT_ref, out_ref, qb_ref):
    i = pl.program_id(0)
    j = pl.program_id(1)

    @pl.when(j == 0)
    def _():
        q = qs_ref[pl.ds(i * _BI, _BI), :].astype(jnp.bfloat16)
        for d in range(_D):
            qb_ref[d] = jnp.broadcast_to(q[:, d:d + 1], (_BI, _BJ))

    accs = [jnp.zeros((_BI, _BJ), jnp.bfloat16) for _ in range(4)]
    for d in range(_D):
        accs[d % 4] = accs[d % 4] + jnp.abs(qb_ref[d] - teT_ref[d:d + 1, :])
    s1 = [accs[0] + accs[1], accs[2] + accs[3]]
    out_ref[...] = -(s1[0] + s1[1]).astype(jnp.float32)


def kernel(ent_pkl, other_emb, W_proj, batch_input_seqs, target_ent_index):
    seq = batch_input_seqs.astype(jnp.int32)
    t_idx = target_ent_index.astype(jnp.int32)
    all_idx = jnp.concatenate([seq[:, 0], seq[:, 1], t_idx])
    is_ent = (all_idx >= 1) & (all_idx <= _NUM_ENT)
    feat_idx = jnp.where(is_ent, all_idx - 1, 0).astype(jnp.int32)
    q_idx = all_idx[:_NQ]
    spec_idx = jnp.where(q_idx == 0, 0, q_idx - _NUM_ENT)
    spec_idx = jnp.clip(spec_idx, 0, _NUM_REL + 2)

    feats = _sc_gather(ent_pkl, feat_idx)

    other_sm = jnp.pad(other_emb, ((0, _NSPAD - (_NUM_REL + 3)), (0, 0)))
    spec_f = spec_idx.astype(jnp.int32)[:, None]

    mask = is_ent.astype(jnp.float32)
    mask_q = mask[:_NQ, None]
    mask_tT = mask[None, _NQ:]
    other0 = other_emb[0][:, None]

    qs, teT = pl.pallas_call(
        _prep_body,
        out_shape=(
            jax.ShapeDtypeStruct((_B, _D), jnp.float32),
            jax.ShapeDtypeStruct((_D, _B), jnp.bfloat16),
        ),
    )(feats, W_proj, other_sm, spec_f, mask_q, mask_tT, other0)

    score = pl.pallas_call(
        _score_body,
        grid=(_B // _BI, _B // _BJ),
        in_specs=[
            pl.BlockSpec((_B, _D), lambda i, j: (0, 0)),
            pl.BlockSpec((_D, _BJ), lambda i, j: (0, j)),
        ],
        out_specs=pl.BlockSpec((_BI, _BJ), lambda i, j: (i, j)),
        out_shape=jax.ShapeDtypeStruct((_B, _B), jnp.float32),
        scratch_shapes=[pltpu.VMEM((2 * _D, _BI, _BJ), jnp.bfloat16)],
    )(qs, teT)
    return score


# speedup vs baseline: 3.6862x; 1.0003x over previous
import functools

import jax
import jax.numpy as jnp
from jax import lax
from jax.experimental import pallas as pl
from jax.experimental.pallas import tpu as pltpu
from jax.experimental.pallas import tpu_sc as plsc

_NUM_ENT = 100000
_NUM_REL = 200
_D = 64
_F = 128
_B = 1024
_NIDX = 3 * _B
_NQ = 2 * _B
_NSPAD = 208


def _sc_gather(ent_pkl, feat_idx):
    info = plsc.get_sparse_core_info()
    nc, ns = info.num_cores, info.num_subcores
    nw = nc * ns
    bpw_f = _NIDX // nw
    mesh = plsc.VectorSubcoreMesh(core_axis_name="c", subcore_axis_name="s")

    @functools.partial(
        pl.kernel,
        mesh=mesh,
        compiler_params=pltpu.CompilerParams(use_tc_tiling_on_sc=True),
        out_type=jax.ShapeDtypeStruct((_NIDX, _F), jnp.float32),
        scratch_types=[
            pltpu.VMEM((bpw_f,), jnp.int32),
            pltpu.VMEM((bpw_f, _F), jnp.float32),
            pltpu.SemaphoreType.DMA,
        ],
    )
    def gather_k(ent_hbm, fidx_hbm, feats_out, fidx_v, frows_v, sem_f):
        wid = lax.axis_index("s") * nc + lax.axis_index("c")
        base_f = wid * bpw_f
        pltpu.sync_copy(fidx_hbm.at[pl.ds(base_f, bpw_f)], fidx_v)
        pltpu.async_copy(ent_hbm.at[fidx_v], frows_v, sem_f).wait()
        pltpu.sync_copy(frows_v, feats_out.at[pl.ds(base_f, bpw_f)])

    return gather_k(ent_pkl, feat_idx)


def _prep_body(ent_hbm, fidx_ref, w_ref, other_sm_ref, spec_f_ref, mask_q_ref,
               mask_tT_ref, other0_ref, qs_ref, teT_ref, feats_ref, sem):
    def issue(k, carry):
        r = fidx_ref[k]
        pltpu.make_async_copy(ent_hbm.at[pl.ds(r, 1)],
                              feats_ref.at[pl.ds(k, 1)], sem).start()
        return carry

    lax.fori_loop(0, _NIDX, issue, 0, unroll=8)
    pltpu.make_async_copy(ent_hbm.at[pl.ds(0, _NIDX)], feats_ref, sem).wait()

    w = w_ref[...]
    sid = spec_f_ref[...]
    lanes = lax.broadcasted_iota(jnp.int32, (1, _NSPAD), 1)
    onehot = (sid == lanes).astype(jnp.float32)
    specs = jnp.dot(onehot, other_sm_ref[...],
                    preferred_element_type=jnp.float32)
    proj_q = jnp.dot(feats_ref[:_NQ, :], w,
                     preferred_element_type=jnp.float32)
    mq = mask_q_ref[...]
    rows_q = mq * proj_q + (1.0 - mq) * specs
    nrm = jnp.sqrt(jnp.sum(rows_q * rows_q, axis=-1, keepdims=True))
    qn = rows_q / jnp.maximum(nrm, 1e-12)
    qs_ref[...] = qn[:_B] + qn[_B:]
    projT_t = lax.dot_general(
        w, feats_ref[_NQ:, :], (((0,), (1,)), ((), ())),
        preferred_element_type=jnp.float32)
    mt = mask_tT_ref[...]
    teT = mt * projT_t + (1.0 - mt) * other0_ref[...]
    teT_ref[...] = teT.astype(jnp.bfloat16)


_BI = 128
_BJ = 128


_NBI = _B // _BI
_NBJ = _B // _BJ
_DPG = _D // _NBJ


def _score_body(qs_ref, teT_ref, out_ref, qb_ref):
    i = pl.program_id(0)
    j = pl.program_id(1)

    @pl.when(j == 0)
    def _():
        q = qs_ref[pl.ds(i * _BI, _BI), :].astype(jnp.bfloat16)
        for d in range(_D):
            qb_ref[d] = jnp.broadcast_to(q[:, d:d + 1], (_BI, _BJ))

    accs = [jnp.zeros((_BI, _BJ), jnp.bfloat16) for _ in range(4)]
    for d in range(_D):
        accs[d % 4] = accs[d % 4] + jnp.abs(qb_ref[d] - teT_ref[d:d + 1, :])
    s1 = [accs[0] + accs[1], accs[2] + accs[3]]
    out_ref[...] = -(s1[0] + s1[1]).astype(jnp.float32)


def kernel(ent_pkl, other_emb, W_proj, batch_input_seqs, target_ent_index):
    seq = batch_input_seqs.astype(jnp.int32)
    t_idx = target_ent_index.astype(jnp.int32)
    all_idx = jnp.concatenate([seq[:, 0], seq[:, 1], t_idx])
    is_ent = (all_idx >= 1) & (all_idx <= _NUM_ENT)
    feat_idx = jnp.where(is_ent, all_idx - 1, 0).astype(jnp.int32)
    q_idx = all_idx[:_NQ]
    spec_idx = jnp.where(q_idx == 0, 0, q_idx - _NUM_ENT)
    spec_idx = jnp.clip(spec_idx, 0, _NUM_REL + 2)

    other_sm = jnp.pad(other_emb, ((0, _NSPAD - (_NUM_REL + 3)), (0, 0)))
    spec_f = spec_idx.astype(jnp.int32)[:, None]

    mask = is_ent.astype(jnp.float32)
    mask_q = mask[:_NQ, None]
    mask_tT = mask[None, _NQ:]
    other0 = other_emb[0][:, None]

    qs, teT = pl.pallas_call(
        _prep_body,
        in_specs=[
            pl.BlockSpec(memory_space=pl.ANY),
            pl.BlockSpec(memory_space=pltpu.SMEM),
            pl.BlockSpec(memory_space=pltpu.VMEM),
            pl.BlockSpec(memory_space=pltpu.VMEM),
            pl.BlockSpec(memory_space=pltpu.VMEM),
            pl.BlockSpec(memory_space=pltpu.VMEM),
            pl.BlockSpec(memory_space=pltpu.VMEM),
            pl.BlockSpec(memory_space=pltpu.VMEM),
        ],
        out_shape=(
            jax.ShapeDtypeStruct((_B, _D), jnp.float32),
            jax.ShapeDtypeStruct((_D, _B), jnp.bfloat16),
        ),
        scratch_shapes=[
            pltpu.VMEM((_NIDX, _F), jnp.float32),
            pltpu.SemaphoreType.DMA,
        ],
    )(ent_pkl, feat_idx, W_proj, other_sm, spec_f, mask_q, mask_tT, other0)

    score = pl.pallas_call(
        _score_body,
        grid=(_B // _BI, _B // _BJ),
        in_specs=[
            pl.BlockSpec((_B, _D), lambda i, j: (0, 0)),
            pl.BlockSpec((_D, _BJ), lambda i, j: (0, j)),
        ],
        out_specs=pl.BlockSpec((_BI, _BJ), lambda i, j: (i, j)),
        out_shape=jax.ShapeDtypeStruct((_B, _B), jnp.float32),
        scratch_shapes=[pltpu.VMEM((2 * _D, _BI, _BJ), jnp.bfloat16)],
    )(qs, teT)
    return score


# speedup vs baseline: 4.3430x; 1.1782x over previous
import jax
import jax.numpy as jnp
from jax import lax
from jax.experimental import pallas as pl
from jax.experimental.pallas import tpu as pltpu

_NUM_ENT = 100000
_NUM_REL = 200
_D = 64
_F = 128
_B = 1024
_NIDX = 3 * _B
_NQ = 2 * _B
_NSPAD = 208

_BJ = 128
_NBJ = _B // _BJ
_BI = 128
_NBI = _B // _BI


def _body(ent_hbm, fidx_ref, w_ref, other_sm_ref, spec_f_ref, mask_q_ref,
          mask_tT_ref, other0_ref, out_ref,
          feats_ref, qs_ref, teT_ref, qb_ref, sem):
    j = pl.program_id(0)

    @pl.when(j == 0)
    def _():
        def issue(k, carry):
            r = fidx_ref[k]
            pltpu.make_async_copy(ent_hbm.at[pl.ds(r, 1)],
                                  feats_ref.at[pl.ds(k, 1)], sem).start()
            return carry

        lax.fori_loop(0, _NIDX, issue, 0, unroll=8)
        pltpu.make_async_copy(ent_hbm.at[pl.ds(0, _NIDX)], feats_ref,
                              sem).wait()

        w = w_ref[...]
        sid = spec_f_ref[...]
        lanes = lax.broadcasted_iota(jnp.int32, (1, _NSPAD), 1)
        onehot = (sid == lanes).astype(jnp.float32)
        specs = jnp.dot(onehot, other_sm_ref[...],
                        preferred_element_type=jnp.float32)
        proj_q = jnp.dot(feats_ref[:_NQ, :], w,
                         preferred_element_type=jnp.float32)
        mq = mask_q_ref[...]
        rows_q = mq * proj_q + (1.0 - mq) * specs
        nrm = jnp.sqrt(jnp.sum(rows_q * rows_q, axis=-1, keepdims=True))
        qn = rows_q / jnp.maximum(nrm, 1e-12)
        qs = qn[:_B] + qn[_B:]
        qs_ref[...] = qs
        projT_t = lax.dot_general(
            w, feats_ref[_NQ:, :], (((0,), (1,)), ((), ())),
            preferred_element_type=jnp.float32)
        mt = mask_tT_ref[...]
        teT = (mt * projT_t
               + (1.0 - mt) * other0_ref[...]).astype(jnp.bfloat16)
        for jj in range(_NBJ):
            teT_ref[jj] = teT[:, jj * _BJ:(jj + 1) * _BJ]
        qsb = qs.astype(jnp.bfloat16)
        for d in range(_D):
            for i in range(_NBI):
                qb_ref[d, i] = jnp.broadcast_to(
                    qsb[i * _BI:(i + 1) * _BI, d:d + 1], (_BI, _BJ))

    tj = teT_ref[j]
    for i in range(_NBI):
        accs = [jnp.zeros((_BI, _BJ), jnp.bfloat16) for _ in range(4)]
        for d in range(_D):
            accs[d % 4] = accs[d % 4] + jnp.abs(qb_ref[d, i] - tj[d:d + 1, :])
        s1 = [accs[0] + accs[1], accs[2] + accs[3]]
        out_ref[pl.ds(i * _BI, _BI), :] = -(s1[0] + s1[1]).astype(jnp.float32)


def kernel(ent_pkl, other_emb, W_proj, batch_input_seqs, target_ent_index):
    seq = batch_input_seqs.astype(jnp.int32)
    t_idx = target_ent_index.astype(jnp.int32)
    all_idx = jnp.concatenate([seq[:, 0], seq[:, 1], t_idx])
    is_ent = (all_idx >= 1) & (all_idx <= _NUM_ENT)
    feat_idx = jnp.where(is_ent, all_idx - 1, 0).astype(jnp.int32)
    q_idx = all_idx[:_NQ]
    spec_idx = jnp.where(q_idx == 0, 0, q_idx - _NUM_ENT)
    spec_idx = jnp.clip(spec_idx, 0, _NUM_REL + 2)

    other_sm = jnp.pad(other_emb, ((0, _NSPAD - (_NUM_REL + 3)), (0, 0)))
    spec_f = spec_idx.astype(jnp.int32)[:, None]

    mask = is_ent.astype(jnp.float32)
    mask_q = mask[:_NQ, None]
    mask_tT = mask[None, _NQ:]
    other0 = other_emb[0][:, None]

    return pl.pallas_call(
        _body,
        grid=(_NBJ,),
        in_specs=[
            pl.BlockSpec(memory_space=pl.ANY),
            pl.BlockSpec(memory_space=pltpu.SMEM),
            pl.BlockSpec((_F, _D), lambda j: (0, 0)),
            pl.BlockSpec((_NSPAD, _D), lambda j: (0, 0)),
            pl.BlockSpec((_NQ, 1), lambda j: (0, 0)),
            pl.BlockSpec((_NQ, 1), lambda j: (0, 0)),
            pl.BlockSpec((1, _B), lambda j: (0, 0)),
            pl.BlockSpec((_D, 1), lambda j: (0, 0)),
        ],
        out_specs=pl.BlockSpec((_B, _BJ), lambda j: (0, j)),
        out_shape=jax.ShapeDtypeStruct((_B, _B), jnp.float32),
        scratch_shapes=[
            pltpu.VMEM((_NIDX, _F), jnp.float32),
            pltpu.VMEM((_B, _D), jnp.float32),
            pltpu.VMEM((_NBJ, _D, _BJ), jnp.bfloat16),
            pltpu.VMEM((_D, _NBI, _BI, _BJ), jnp.bfloat16),
            pltpu.SemaphoreType.DMA,
        ],
    )(ent_pkl, feat_idx, W_proj, other_sm, spec_f, mask_q, mask_tT, other0)
